# baseline (speedup 1.0000x reference)
import jax
import jax.numpy as jnp
from jax.experimental import pallas as pl
from jax.experimental.pallas import tpu as pltpu

_BM = 200
_NBUF = 4


def _gcn_body(x_hbm, wt_hbm, b_hbm, adj_hbm, out_hbm,
              x_ref, wt_ref, b_ref, h_ref, bufs_ref, obuf_ref,
              in_sems, out_sems, pre_sems):
    N, d = h_ref.shape
    nchunks = N // _BM

    xcopy = pltpu.make_async_copy(x_hbm, x_ref, pre_sems.at[0])
    wcopy = pltpu.make_async_copy(wt_hbm, wt_ref, pre_sems.at[1])
    bcopy = pltpu.make_async_copy(b_hbm, b_ref, pre_sems.at[2])
    xcopy.start()
    wcopy.start()
    bcopy.start()
    for s in range(_NBUF):
        pltpu.make_async_copy(
            adj_hbm.at[pl.ds(s * _BM, _BM), :], bufs_ref.at[s], in_sems.at[s]
        ).start()
    xcopy.wait()
    wcopy.wait()
    bcopy.wait()
    h_ref[...] = (
        jnp.dot(x_ref[...], wt_ref[...], preferred_element_type=jnp.float32)
        + b_ref[...]
    )

    def step(i, carry):
        slot = jax.lax.rem(i, _NBUF)
        oslot = jax.lax.rem(i, 2)
        pltpu.make_async_copy(
            adj_hbm.at[pl.ds(i * _BM, _BM), :], bufs_ref.at[slot],
            in_sems.at[slot],
        ).wait()

        @pl.when(i >= 2)
        def _():
            pltpu.make_async_copy(
                obuf_ref.at[oslot], out_hbm.at[pl.ds((i - 2) * _BM, _BM), :],
                out_sems.at[oslot],
            ).wait()

        obuf_ref[oslot] = jnp.dot(
            bufs_ref[slot], h_ref[...], preferred_element_type=jnp.float32
        )
        pltpu.make_async_copy(
            obuf_ref.at[oslot], out_hbm.at[pl.ds(i * _BM, _BM), :],
            out_sems.at[oslot],
        ).start()

        @pl.when(i + _NBUF < nchunks)
        def _():
            pltpu.make_async_copy(
                adj_hbm.at[pl.ds((i + _NBUF) * _BM, _BM), :],
                bufs_ref.at[slot], in_sems.at[slot],
            ).start()

        return carry

    jax.lax.fori_loop(0, nchunks, step, 0)

    for i in (nchunks - 2, nchunks - 1):
        pltpu.make_async_copy(
            obuf_ref.at[i % 2], out_hbm.at[pl.ds(i * _BM, _BM), :],
            out_sems.at[i % 2],
        ).wait()


def kernel(x, adj, W, b, is_sparse):
    N, d = x.shape
    out = pl.pallas_call(
        _gcn_body,
        in_specs=[pl.BlockSpec(memory_space=pl.ANY)] * 4,
        out_specs=pl.BlockSpec(memory_space=pl.ANY),
        out_shape=jax.ShapeDtypeStruct((N, d), jnp.float32),
        scratch_shapes=[
            pltpu.VMEM((N, d), jnp.float32),
            pltpu.VMEM((d, d), jnp.float32),
            pltpu.VMEM((1, d), jnp.float32),
            pltpu.VMEM((N, d), jnp.float32),
            pltpu.VMEM((_NBUF, _BM, N), jnp.float32),
            pltpu.VMEM((2, _BM, d), jnp.float32),
            pltpu.SemaphoreType.DMA((_NBUF,)),
            pltpu.SemaphoreType.DMA((2,)),
            pltpu.SemaphoreType.DMA((3,)),
        ],
    )(x, W.T, b.reshape(1, d), adj)
    return out


# speedup vs baseline: 1.0052x; 1.0052x over previous
import jax
import jax.numpy as jnp
from jax.experimental import pallas as pl
from jax.experimental.pallas import tpu as pltpu


def _gcn_body(x_ref, wt_ref, b_ref, adj_ref, out_ref, h_ref):
    @pl.when(pl.program_id(0) == 0)
    def _():
        h_ref[...] = (
            jnp.dot(x_ref[...], wt_ref[...], preferred_element_type=jnp.float32)
            + b_ref[...]
        )

    out_ref[...] = jnp.dot(
        adj_ref[...], h_ref[...], preferred_element_type=jnp.float32
    )


def kernel(x, adj, W, b, is_sparse):
    N, d = x.shape
    BM = 400 if N % 400 == 0 else N
    out = pl.pallas_call(
        _gcn_body,
        grid=(N // BM,),
        in_specs=[
            pl.BlockSpec((N, d), lambda m: (0, 0)),
            pl.BlockSpec((d, d), lambda m: (0, 0)),
            pl.BlockSpec((1, d), lambda m: (0, 0)),
            pl.BlockSpec((BM, N), lambda m: (m, 0)),
        ],
        out_specs=pl.BlockSpec((BM, d), lambda m: (m, 0)),
        out_shape=jax.ShapeDtypeStruct((N, d), jnp.float32),
        scratch_shapes=[pltpu.VMEM((N, d), jnp.float32)],
    )(x, W.T, b.reshape(1, d), adj)
    return out
